# baseline (speedup 1.0000x reference)
import functools

import jax
import jax.numpy as jnp
from jax.experimental import pallas as pl
from jax.experimental.pallas import tpu as pltpu

B, T, C, H = 1, 512, 1024, 512
G, EPG, K = 8, 8, 2
E = G * EPG
S = B * T
NEG = -1e30


def _silu(x):
    return x * jax.nn.sigmoid(x)


def _routing_kernel(x_ref, ln_scale_ref, ln_bias_ref, shared_in_ref,
                    shared_out_ref, shared_out_b_ref, expert_in_ref,
                    expert_out_b_ref, group_gate_ref, expert_gate_ref,
                    group_bias_ref, expert_bias_ref,
                    base_ref, h_ref, coef_ref):
    flat = x_ref[...]
    mu = jnp.mean(flat, axis=-1, keepdims=True)
    var = jnp.mean((flat - mu) ** 2, axis=-1, keepdims=True)
    flat = (flat - mu) * jax.lax.rsqrt(var + 1e-5)
    flat = flat * ln_scale_ref[...] + ln_bias_ref[...]

    hs = jnp.dot(flat, shared_in_ref[...], preferred_element_type=jnp.float32)
    a = hs[:, :H]
    b = hs[:, H:]
    h_shared = _silu(a) * b
    out_shared = (jnp.dot(h_shared, shared_out_ref[...],
                          preferred_element_type=jnp.float32)
                  + shared_out_b_ref[...])

    g_logits = (jnp.dot(flat, group_gate_ref[...],
                        preferred_element_type=jnp.float32)
                + group_bias_ref[...])
    g_max = jnp.max(g_logits, axis=-1, keepdims=True)
    g_iota = jax.lax.broadcasted_iota(jnp.int32, (S, G), 1)
    group_idx = jnp.min(jnp.where(g_logits == g_max, g_iota, G),
                        axis=-1, keepdims=True)

    e_logits = (jnp.dot(flat, expert_gate_ref[...],
                        preferred_element_type=jnp.float32)
                + expert_bias_ref[...])
    e_iota = jax.lax.broadcasted_iota(jnp.int32, (S, E), 1)
    in_group = (e_iota // EPG) == group_idx
    e_masked = jnp.where(in_group, e_logits, NEG)
    m = jnp.max(e_masked, axis=-1, keepdims=True)
    p = jnp.exp(e_masked - m)
    p = p / jnp.sum(p, axis=-1, keepdims=True)

    v1 = jnp.max(p, axis=-1, keepdims=True)
    i1 = jnp.min(jnp.where(p == v1, e_iota, E), axis=-1, keepdims=True)
    p2 = jnp.where(e_iota == i1, -1.0, p)
    v2 = jnp.max(p2, axis=-1, keepdims=True)
    i2 = jnp.min(jnp.where(p2 == v2, e_iota, E), axis=-1, keepdims=True)
    denom = v1 + v2 + 1e-8
    w1 = v1 / denom
    w2 = v2 / denom

    sel1 = (e_iota == i1).astype(jnp.float32)
    sel2 = (e_iota == i2).astype(jnp.float32)
    coef = sel1 * w1 + sel2 * w2
    coef_ref[...] = coef

    bias_routed = jnp.dot(sel1 + sel2, expert_out_b_ref[...],
                          preferred_element_type=jnp.float32)
    base_ref[...] = out_shared + bias_routed

    he = jnp.dot(flat, expert_in_ref[...], preferred_element_type=jnp.float32)
    h_ref[...] = _silu(he[:, :H]) * he[:, H:]


NSTREAM = 8
NCHUNK = 4
CCH = C // NCHUNK
ESTEP = E // NSTREAM


def _expert_kernel(base_ref, h_ref, coef_ref, *rest):
    w2_refs = rest[:NSTREAM * NCHUNK]
    out_ref = rest[NSTREAM * NCHUNK]
    s = pl.program_id(0)
    eiota = jax.lax.broadcasted_iota(jnp.int32, (E, NSTREAM), 0)
    j = jax.lax.broadcasted_iota(jnp.int32, (E, NSTREAM), 1)
    onehot = (eiota == j * ESTEP + s).astype(jnp.float32)
    cs = jnp.dot(coef_ref[...], onehot, preferred_element_type=jnp.float32)
    h = h_ref[...]
    for jc in range(NCHUNK):
        acc = 0.0
        for i in range(NSTREAM):
            scaled = (h * cs[:, i:i + 1]).astype(jnp.bfloat16)
            acc += jax.lax.dot_general(
                scaled, w2_refs[i * NCHUNK + jc][0].astype(jnp.bfloat16),
                dimension_numbers=(((1,), (1,)), ((), ())),
                preferred_element_type=jnp.float32)

        lo = jc * CCH

        @pl.when(s == 0)
        def _(lo=lo, acc=acc):
            out_ref[:, lo:lo + CCH] = base_ref[:, lo:lo + CCH] + acc

        @pl.when(s > 0)
        def _(lo=lo, acc=acc):
            out_ref[:, lo:lo + CCH] = out_ref[:, lo:lo + CCH] + acc


def kernel(x, ln_scale, ln_bias, shared_in_w, shared_out_w, shared_out_b,
           expert_in_w, expert_out_w, expert_out_b, group_gate_w,
           expert_gate_w, group_bias_buf, expert_bias_buf):
    flat = x.reshape(S, C)

    base, h_expert, coef = pl.pallas_call(
        _routing_kernel,
        out_shape=[
            jax.ShapeDtypeStruct((S, C), jnp.float32),
            jax.ShapeDtypeStruct((S, H), jnp.float32),
            jax.ShapeDtypeStruct((S, E), jnp.float32),
        ],
    )(flat, ln_scale.reshape(1, C), ln_bias.reshape(1, C), shared_in_w,
      shared_out_w, shared_out_b.reshape(1, C), expert_in_w, expert_out_b,
      group_gate_w, expert_gate_w, group_bias_buf.reshape(1, G),
      expert_bias_buf.reshape(1, E))

    out = pl.pallas_call(
        _expert_kernel,
        grid=(ESTEP,),
        in_specs=[
            pl.BlockSpec((S, C), lambda s: (0, 0)),
            pl.BlockSpec((S, H), lambda s: (0, 0)),
            pl.BlockSpec((S, E), lambda s: (0, 0)),
        ] + [pl.BlockSpec((1, CCH, H),
                          lambda s, i=i, jc=jc: (i * ESTEP + s, jc, 0))
             for i in range(NSTREAM) for jc in range(NCHUNK)],
        out_specs=pl.BlockSpec((S, C), lambda s: (0, 0)),
        out_shape=jax.ShapeDtypeStruct((S, C), jnp.float32),
        compiler_params=pltpu.CompilerParams(
            vmem_limit_bytes=110 * 1024 * 1024),
    )(base, h_expert, coef, *([expert_out_w] * (NSTREAM * NCHUNK)))

    return out.reshape(B, T, C)


# speedup vs baseline: 1.6470x; 1.6470x over previous
import jax
import jax.numpy as jnp
from jax.experimental import pallas as pl
from jax.experimental.pallas import tpu as pltpu

B, T, C, H = 1, 512, 1024, 512
G, EPG, K = 8, 8, 2
E = G * EPG
S = B * T
NEG = -1e30

NSTREAM = 8
NCHUNK = 2
CCH = C // NCHUNK
ESTEP = E // NSTREAM


def _silu(x):
    return x * jax.nn.sigmoid(x)


def _fused_kernel(x_ref, ln_scale_ref, ln_bias_ref, shared_in_ref,
                  shared_out_ref, shared_out_b_ref, expert_in_ref,
                  expert_out_b_ref, group_gate_ref, expert_gate_ref,
                  group_bias_ref, expert_bias_ref, *rest):
    w2_refs = rest[:NSTREAM * NCHUNK]
    out_ref = rest[NSTREAM * NCHUNK]
    h_ref = rest[NSTREAM * NCHUNK + 1]
    coef_ref = rest[NSTREAM * NCHUNK + 2]
    s = pl.program_id(0)

    @pl.when(s == 0)
    def _():
        flat = x_ref[...]
        mu = jnp.mean(flat, axis=-1, keepdims=True)
        var = jnp.mean((flat - mu) ** 2, axis=-1, keepdims=True)
        flat = (flat - mu) * jax.lax.rsqrt(var + 1e-5)
        flat = flat * ln_scale_ref[...] + ln_bias_ref[...]

        hs = jnp.dot(flat, shared_in_ref[...],
                     preferred_element_type=jnp.float32)
        h_shared = _silu(hs[:, :H]) * hs[:, H:]
        out_shared = (jnp.dot(h_shared, shared_out_ref[...],
                              preferred_element_type=jnp.float32)
                      + shared_out_b_ref[...])

        g_logits = (jnp.dot(flat, group_gate_ref[...],
                            preferred_element_type=jnp.float32)
                    + group_bias_ref[...])
        g_max = jnp.max(g_logits, axis=-1, keepdims=True)
        g_iota = jax.lax.broadcasted_iota(jnp.int32, (S, G), 1)
        group_idx = jnp.min(jnp.where(g_logits == g_max, g_iota, G),
                            axis=-1, keepdims=True)

        e_logits = (jnp.dot(flat, expert_gate_ref[...],
                            preferred_element_type=jnp.float32)
                    + expert_bias_ref[...])
        e_iota = jax.lax.broadcasted_iota(jnp.int32, (S, E), 1)
        in_group = (e_iota // EPG) == group_idx
        e_masked = jnp.where(in_group, e_logits, NEG)
        m = jnp.max(e_masked, axis=-1, keepdims=True)
        p = jnp.exp(e_masked - m)
        p = p / jnp.sum(p, axis=-1, keepdims=True)

        v1 = jnp.max(p, axis=-1, keepdims=True)
        i1 = jnp.min(jnp.where(p == v1, e_iota, E), axis=-1, keepdims=True)
        p2 = jnp.where(e_iota == i1, -1.0, p)
        v2 = jnp.max(p2, axis=-1, keepdims=True)
        i2 = jnp.min(jnp.where(p2 == v2, e_iota, E), axis=-1, keepdims=True)
        denom = v1 + v2 + 1e-8

        sel1 = (e_iota == i1).astype(jnp.float32)
        sel2 = (e_iota == i2).astype(jnp.float32)
        coef_ref[...] = sel1 * (v1 / denom) + sel2 * (v2 / denom)

        bias_routed = jnp.dot(sel1 + sel2, expert_out_b_ref[...],
                              preferred_element_type=jnp.float32)
        out_ref[...] = out_shared + bias_routed

        he = jnp.dot(flat, expert_in_ref[...],
                     preferred_element_type=jnp.float32)
        h_ref[...] = _silu(he[:, :H]) * he[:, H:]

    eiota = jax.lax.broadcasted_iota(jnp.int32, (E, NSTREAM), 0)
    j = jax.lax.broadcasted_iota(jnp.int32, (E, NSTREAM), 1)
    onehot = (eiota == j * ESTEP + s).astype(jnp.float32)
    cs = jnp.dot(coef_ref[...], onehot, preferred_element_type=jnp.float32)
    h = h_ref[...]
    for jc in range(NCHUNK):
        acc = 0.0
        for i in range(NSTREAM):
            scaled = (h * cs[:, i:i + 1]).astype(jnp.bfloat16)
            acc += jax.lax.dot_general(
                scaled, w2_refs[i * NCHUNK + jc][0].astype(jnp.bfloat16),
                dimension_numbers=(((1,), (1,)), ((), ())),
                preferred_element_type=jnp.float32)
        lo = jc * CCH
        out_ref[:, lo:lo + CCH] = out_ref[:, lo:lo + CCH] + acc


def kernel(x, ln_scale, ln_bias, shared_in_w, shared_out_w, shared_out_b,
           expert_in_w, expert_out_w, expert_out_b, group_gate_w,
           expert_gate_w, group_bias_buf, expert_bias_buf):
    flat = x.reshape(S, C)
    full = lambda shape: pl.BlockSpec(shape, lambda s: tuple(0 for _ in shape))

    out = pl.pallas_call(
        _fused_kernel,
        grid=(ESTEP,),
        in_specs=[
            full((S, C)), full((1, C)), full((1, C)), full((C, 2 * H)),
            full((H, C)), full((1, C)), full((C, 2 * H)), full((E, C)),
            full((C, G)), full((C, E)), full((1, G)), full((1, E)),
        ] + [pl.BlockSpec((1, CCH, H),
                          lambda s, i=i, jc=jc: (i * ESTEP + s, jc, 0))
             for i in range(NSTREAM) for jc in range(NCHUNK)],
        out_specs=full((S, C)),
        out_shape=jax.ShapeDtypeStruct((S, C), jnp.float32),
        scratch_shapes=[
            pltpu.VMEM((S, H), jnp.float32),
            pltpu.VMEM((S, E), jnp.float32),
        ],
        compiler_params=pltpu.CompilerParams(
            vmem_limit_bytes=100 * 1024 * 1024),
    )(flat, ln_scale.reshape(1, C), ln_bias.reshape(1, C), shared_in_w,
      shared_out_w, shared_out_b.reshape(1, C), expert_in_w, expert_out_b,
      group_gate_w, expert_gate_w, group_bias_buf.reshape(1, G),
      expert_bias_buf.reshape(1, E), *([expert_out_w] * (NSTREAM * NCHUNK)))

    return out.reshape(B, T, C)
